# baseline (speedup 1.0000x reference)
import functools

import jax
import jax.numpy as jnp
import numpy as np
from jax import lax
from jax.experimental import pallas as pl
from jax.experimental.pallas import tpu as pltpu
from jax.experimental.pallas import tpu_sc as plsc

_MAX_LEN = 200
_EMB_DIM = 64


def _make_pos_encoding():
    pos = np.expand_dims(np.arange(_MAX_LEN), 1)
    pe = pos / np.power(1000, 2 * np.expand_dims(np.arange(_EMB_DIM), 0) / _EMB_DIM)
    pe[:, 0::2] = np.sin(pe[:, 0::2])
    pe[:, 1::2] = np.cos(pe[:, 1::2])
    return jnp.asarray(pe, dtype=jnp.float32)


_PE = _make_pos_encoding()

_NUM_CORES = 2
_NUM_SUBCORES = 16
_NW = _NUM_CORES * _NUM_SUBCORES
_ROWS_PER_CHUNK = 4


@functools.partial(jax.jit, static_argnames=("batch", "seq"))
def _embed_lookup(x_flat, table, pe, *, batch, seq):
    n_rows = batch * seq
    rows_per_w = n_rows // _NW
    batch_per_w = batch // _NW
    chunk = _ROWS_PER_CHUNK * seq
    n_steps = batch_per_w // _ROWS_PER_CHUNK

    mesh = plsc.VectorSubcoreMesh(core_axis_name="c", subcore_axis_name="s")

    @functools.partial(
        pl.kernel,
        out_type=jax.ShapeDtypeStruct((n_rows, _EMB_DIM), jnp.float32),
        mesh=mesh,
        compiler_params=pltpu.CompilerParams(use_tc_tiling_on_sc=False),
        scratch_types=[
            pltpu.VMEM_SHARED((seq, _EMB_DIM), jnp.float32),
            pltpu.VMEM((chunk,), jnp.int32),
            pltpu.VMEM((chunk, _EMB_DIM), jnp.float32),
        ],
    )
    def k(x_hbm, table_hbm, pe_hbm, out_hbm, pe_sh, idx_v, rows_v):
        sid = lax.axis_index("s")
        wid = sid * _NUM_CORES + lax.axis_index("c")
        base = wid * rows_per_w

        @pl.when(sid == 0)
        def _():
            pltpu.sync_copy(pe_hbm, pe_sh)

        plsc.subcore_barrier()

        @pl.loop(0, n_steps)
        def _(step):
            off = base + step * chunk
            pltpu.sync_copy(x_hbm.at[pl.ds(off, chunk)], idx_v)
            for c in range(_ROWS_PER_CHUNK):
                pltpu.sync_copy(pe_sh, rows_v.at[pl.ds(c * seq, seq)])
            pltpu.sync_copy(table_hbm.at[idx_v], rows_v, add=True)
            pltpu.sync_copy(rows_v, out_hbm.at[pl.ds(off, chunk)])

    return k(x_flat, table, pe)


def kernel(x, table):
    batch, seq = x.shape
    out = _embed_lookup(x.reshape(-1), table, _PE, batch=batch, seq=seq)
    return out.reshape(batch, seq, _EMB_DIM)
